# baseline (speedup 1.0000x reference)
import functools

import jax
import jax.numpy as jnp
from jax import lax
from jax.experimental import pallas as pl
from jax.experimental.pallas import tpu as pltpu
from jax.experimental.pallas import tpu_sc as plsc

BATCH = 4096
SEQ = 200
DIM = 64
CHUNK = 100
NROWS = BATCH * SEQ // CHUNK
NW = 32
CPW = NROWS // NW
BLK = 16
NBLK = CPW // BLK


def _build():
    mesh = plsc.VectorSubcoreMesh(core_axis_name="c", subcore_axis_name="s")

    @functools.partial(
        pl.kernel,
        mesh=mesh,
        out_type=jax.ShapeDtypeStruct((NROWS, CHUNK, DIM), jnp.float32),
        scratch_types=[
            pltpu.VMEM((BLK, CHUNK), jnp.int32),
            pltpu.VMEM((SEQ, DIM), jnp.float32),
            pltpu.VMEM((CHUNK, DIM), jnp.float32),
            pltpu.SemaphoreType.DMA,
        ],
        compiler_params=pltpu.CompilerParams(use_tc_tiling_on_sc=False),
    )
    def emb(ids_hbm, table_hbm, pos_hbm, out_hbm, idx_v, pos_v, buf, sem):
        wid = lax.axis_index("s") * 2 + lax.axis_index("c")
        base = wid * CPW
        pltpu.sync_copy(pos_hbm, pos_v)

        def block_body(k, carry):
            r0 = base + k * BLK
            pltpu.sync_copy(ids_hbm.at[pl.ds(r0, BLK)], idx_v)
            for j in range(BLK):
                pltpu.async_copy(table_hbm.at[idx_v.at[j]], buf, sem).wait()
                p0 = (j & 1) * CHUNK

                def row_body(r, c):
                    for d in range(DIM // 16):
                        sl = pl.ds(d * 16, 16)
                        buf[r, sl] = buf[r, sl] + pos_v[p0 + r, sl]
                    return c

                lax.fori_loop(0, CHUNK, row_body, 0)
                pltpu.sync_copy(buf, out_hbm.at[r0 + j])
            return carry

        lax.fori_loop(0, NBLK, block_body, 0)

    return emb


_emb = _build()


def kernel(token_ids, token_table, pos_table):
    ids = token_ids.reshape(NROWS, CHUNK).astype(jnp.int32)
    out = _emb(ids, token_table, pos_table)
    return out.reshape(BATCH, SEQ, DIM)


# speedup vs baseline: 1.2496x; 1.2496x over previous
import functools

import jax
import jax.numpy as jnp
from jax import lax
from jax.experimental import pallas as pl
from jax.experimental.pallas import tpu as pltpu
from jax.experimental.pallas import tpu_sc as plsc

BATCH = 4096
SEQ = 200
DIM = 64
CHUNK = 100
NROWS = BATCH * SEQ // CHUNK
NW = 32
CPW = NROWS // NW
NBUF = 8
AHEAD = 4
UNROLL = NBUF


def _build():
    mesh = plsc.VectorSubcoreMesh(core_axis_name="c", subcore_axis_name="s")

    @functools.partial(
        pl.kernel,
        mesh=mesh,
        out_type=jax.ShapeDtypeStruct((NROWS, CHUNK, DIM), jnp.float32),
        scratch_types=[
            pltpu.VMEM((CPW, CHUNK), jnp.int32),
            pltpu.VMEM((SEQ, DIM), jnp.float32),
            [pltpu.VMEM((CHUNK, DIM), jnp.float32) for _ in range(NBUF)],
            [pltpu.SemaphoreType.DMA for _ in range(NBUF)],
            [pltpu.SemaphoreType.DMA for _ in range(NBUF)],
        ],
        compiler_params=pltpu.CompilerParams(use_tc_tiling_on_sc=False),
    )
    def emb(ids_hbm, table_hbm, pos_hbm, out_hbm, idx_v, pos_v, bufs, gsem, wsem):
        wid = lax.axis_index("s") * 2 + lax.axis_index("c")
        base = wid * CPW
        pltpu.sync_copy(pos_hbm, pos_v)
        pltpu.sync_copy(ids_hbm.at[pl.ds(base, CPW)], idx_v)

        def fire_gather(c, slot):
            pltpu.async_copy(table_hbm.at[idx_v.at[c]], bufs[slot], gsem[slot])

        for q in range(AHEAD):
            fire_gather(q, q)

        def iter_body(i, carry):
            c0 = i * UNROLL
            for q in range(UNROLL):
                c = c0 + q
                cn = c + AHEAD
                slot_n = (q + AHEAD) % NBUF

                def fire_next(cn=cn, slot_n=slot_n):
                    fire_gather(cn, slot_n)

                def wait_then_fire(cn=cn, slot_n=slot_n):
                    pltpu.make_async_copy(
                        bufs[slot_n], out_hbm.at[0], wsem[slot_n]
                    ).wait()
                    fire_gather(cn, slot_n)

                if q + AHEAD < NBUF:
                    @pl.when(cn < CPW)
                    def _():
                        @pl.when(cn >= NBUF)
                        def _():
                            wait_then_fire()

                        @pl.when(cn < NBUF)
                        def _():
                            fire_next()

                else:
                    @pl.when(cn < CPW)
                    def _():
                        wait_then_fire()

                pltpu.make_async_copy(
                    table_hbm.at[idx_v.at[0]], bufs[q], gsem[q]
                ).wait()
                buf = bufs[q]
                p0 = (q & 1) * CHUNK

                @plsc.parallel_loop(0, CHUNK, step=2, unroll=4)
                def row_body(r):
                    for rr in range(2):
                        for d in range(DIM // 16):
                            sl = pl.ds(d * 16, 16)
                            buf[r + rr, sl] = buf[r + rr, sl] + pos_v[p0 + r + rr, sl]

                pltpu.async_copy(buf, out_hbm.at[base + c], wsem[q])
            return carry

        lax.fori_loop(0, CPW // UNROLL, iter_body, 0)
        for q in range(NBUF):
            pltpu.make_async_copy(bufs[q], out_hbm.at[0], wsem[q]).wait()

    return emb


_emb = _build()


def kernel(token_ids, token_table, pos_table):
    ids = token_ids.reshape(NROWS, CHUNK).astype(jnp.int32)
    out = _emb(ids, token_table, pos_table)
    return out.reshape(BATCH, SEQ, DIM)
